# Initial kernel scaffold; baseline (speedup 1.0000x reference)
import jax
import jax.numpy as jnp
from jax.experimental import pallas as pl


def kernel(x0, x1, up_w1, up_b1, up_wdw, up_bdw, up_w2, up_b2, sh_w1, sh_b1, sh_wdw, sh_bdw, sh_w2, sh_b2):
    raise NotImplementedError("write your pallas kernel here")


# speedup vs baseline: 1.7326x; 1.7326x over previous
import numpy as np
import jax
import jax.numpy as jnp
from jax.experimental import pallas as pl
from jax.experimental.pallas import tpu as pltpu

NB = 8


def _boundary_masks(H, W):
    h = np.arange(H)[:, None]
    w = np.arange(W)[None, :]
    out = np.empty((9, H * W), np.float32)
    t = 0
    for dy in (-1, 0, 1):
        for dx in (-1, 0, 1):
            valid = (h + dy >= 0) & (h + dy < H) & (w + dx >= 0) & (w + dx < W)
            out[t] = valid.reshape(-1).astype(np.float32)
            t += 1
    return out


def _interp_matrix(n_in, n_out):
    if n_in == 1:
        return np.ones((n_out, 1), np.float32)
    src = np.arange(n_out, dtype=np.float64) * (n_in - 1) / (n_out - 1)
    i0 = np.minimum(np.floor(src).astype(np.int64), n_in - 2)
    frac = src - i0
    m = np.zeros((n_out, n_in), np.float64)
    m[np.arange(n_out), i0] += 1.0 - frac
    m[np.arange(n_out), i0 + 1] += frac
    return m.astype(np.float32)


def _up2x_matrix(H0, W0):
    A = _interp_matrix(H0, 2 * H0)
    B = _interp_matrix(W0, 2 * W0)
    return np.kron(A, B).T.astype(np.float32)


def _make_body(nb, chA, chB, C, W0, W1, S0, S1):
    def dw3x3(y2d, mask_ref, W, S):
        y3 = y2d.reshape(nb, C, S)
        pad = W + 1
        z = jnp.zeros((nb, C, pad), jnp.float32)
        ypad = jnp.concatenate([z, y3, z], axis=2)
        acc = y3 * mask_ref[4]
        for dy in (-1, 0, 1):
            for dx in (-1, 0, 1):
                if dy == 0 and dx == 0:
                    continue
                t = (dy + 1) * 3 + (dx + 1)
                s = dy * W + dx
                acc = acc + ypad[:, :, pad + s: pad + s + S] * mask_ref[t]
        return acc.reshape(nb * C, S)

    def body(x0_ref, x1_ref,
             wu1_ref, bu1_ref, mu_ref, bud_ref, wu2_ref, bu2_ref,
             ws1_ref, bs1_ref, ms_ref, bsd_ref, ws2_ref, bs2_ref,
             upmat_ref, o_ref):
        f32 = jnp.float32

        x0 = x0_ref[...].reshape(nb * chA, S0)
        y = jnp.dot(wu1_ref[...], x0, preferred_element_type=f32) + bu1_ref[...]
        y = jnp.maximum(y, 0.0)
        y = dw3x3(y, mu_ref, W0, S0) + bud_ref[...]
        y = jnp.dot(wu2_ref[...], y, preferred_element_type=f32) + bu2_ref[...]
        y = jnp.maximum(y, 0.0)
        up = jnp.dot(y, upmat_ref[...], preferred_element_type=f32)

        x1 = x1_ref[...].reshape(nb * chB, S1)
        z = jnp.dot(ws1_ref[...], x1, preferred_element_type=f32) + bs1_ref[...]
        z = jnp.maximum(z, 0.0)
        z = dw3x3(z, ms_ref, W1, S1) + bsd_ref[...]
        z = jnp.dot(ws2_ref[...], z, preferred_element_type=f32) + bs2_ref[...]
        z = jnp.maximum(z, 0.0)

        o_ref[:, :C, :] = up.reshape(nb, C, S1)
        o_ref[:, C:, :] = z.reshape(nb, C, S1)

    return body


def kernel(x0, x1, up_w1, up_b1, up_wdw, up_bdw, up_w2, up_b2,
           sh_w1, sh_b1, sh_wdw, sh_bdw, sh_w2, sh_b2):
    N, chA, H0, W0 = x0.shape
    _, chB, H1, W1 = x1.shape
    C = up_w1.shape[0]
    S0, S1 = H0 * W0, H1 * W1
    nb = NB if N % NB == 0 else 1

    x0f = x0.reshape(N, chA, S0)
    x1f = x1.reshape(N, chB, S1)

    eye = jnp.eye(nb, dtype=jnp.float32)

    def bd(w):
        return jnp.kron(eye, w)

    def tl(b):
        return jnp.tile(b, (nb, 1))

    wu1, wu2 = bd(up_w1), bd(up_w2)
    ws1, ws2 = bd(sh_w1), bd(sh_w2)
    bu1, bud, bu2 = tl(up_b1), tl(up_bdw), tl(up_b2)
    bs1, bsd, bs2 = tl(sh_b1), tl(sh_bdw), tl(sh_b2)

    mu = jnp.asarray(_boundary_masks(H0, W0))[:, None, :] * up_wdw
    ms = jnp.asarray(_boundary_masks(H1, W1))[:, None, :] * sh_wdw
    upmat = jnp.asarray(_up2x_matrix(H0, W0))

    body = _make_body(nb, chA, chB, C, W0, W1, S0, S1)

    def cspec(arr):
        nd = arr.ndim
        return pl.BlockSpec(arr.shape, lambda b, _nd=nd: (0,) * _nd)

    out = pl.pallas_call(
        body,
        out_shape=jax.ShapeDtypeStruct((N, 2 * C, S1), jnp.float32),
        grid=(N // nb,),
        in_specs=[
            pl.BlockSpec((nb, chA, S0), lambda b: (b, 0, 0)),
            pl.BlockSpec((nb, chB, S1), lambda b: (b, 0, 0)),
            cspec(wu1), cspec(bu1), cspec(mu), cspec(bud),
            cspec(wu2), cspec(bu2),
            cspec(ws1), cspec(bs1), cspec(ms), cspec(bsd),
            cspec(ws2), cspec(bs2),
            cspec(upmat),
        ],
        out_specs=pl.BlockSpec((nb, 2 * C, S1), lambda b: (b, 0, 0)),
        compiler_params=pltpu.CompilerParams(
            dimension_semantics=("parallel",),
            vmem_limit_bytes=100 * 1024 * 1024,
        ),
    )(x0f, x1f, wu1, bu1, mu, bud, wu2, bu2, ws1, bs1, ms, bsd, ws2, bs2,
      upmat)

    return out.reshape(N, 2 * C, H1, W1)


# speedup vs baseline: 2.3191x; 1.3385x over previous
import numpy as np
import jax
import jax.numpy as jnp
from jax.experimental import pallas as pl
from jax.experimental.pallas import tpu as pltpu

NB = 8


def _edge_masks(H, W):
    col = np.tile(np.arange(W), H)
    row = np.repeat(np.arange(H), W)
    return np.stack([
        (col > 0), (col < W - 1), (row > 0), (row < H - 1),
    ]).astype(np.float32)


def _interp_matrix(n_in, n_out):
    if n_in == 1:
        return np.ones((n_out, 1), np.float32)
    src = np.arange(n_out, dtype=np.float64) * (n_in - 1) / (n_out - 1)
    i0 = np.minimum(np.floor(src).astype(np.int64), n_in - 2)
    frac = src - i0
    m = np.zeros((n_out, n_in), np.float64)
    m[np.arange(n_out), i0] += 1.0 - frac
    m[np.arange(n_out), i0 + 1] += frac
    return m.astype(np.float32)


def _up2x_matrix(H0, W0):
    A = _interp_matrix(H0, 2 * H0)
    B = _interp_matrix(W0, 2 * W0)
    return np.kron(A, B).T.astype(np.float32)


def _make_body(nb, chA, chB, C, W0, W1, S0, S1):
    def shl(v, k):
        z = jnp.zeros(v.shape[:-1] + (k,), v.dtype)
        return jnp.concatenate([v[..., k:], z], axis=-1)

    def shr(v, k):
        z = jnp.zeros(v.shape[:-1] + (k,), v.dtype)
        return jnp.concatenate([z, v[..., :-k]], axis=-1)

    def dw3x3(y2d, em_ref, wdw_ref, W, S):
        y = y2d.reshape(nb, C, S)
        a_m = shr(y, 1) * em_ref[0].reshape(1, 1, S)
        a_p = shl(y, 1) * em_ref[1].reshape(1, 1, S)
        def comb(t):
            w = wdw_ref[t].reshape(1, C, 1)
            wm = wdw_ref[t - 1].reshape(1, C, 1)
            wp = wdw_ref[t + 1].reshape(1, C, 1)
            return a_m * wm + y * w + a_p * wp
        b_m, b_0, b_p = comb(1), comb(4), comb(7)
        out = (b_0
               + shr(b_m, W) * em_ref[2].reshape(1, 1, S)
               + shl(b_p, W) * em_ref[3].reshape(1, 1, S))
        return out.reshape(nb * C, S)

    def body(x0_ref, x1_ref,
             wu1_ref, bu1_ref, emu_ref, wdu_ref, bud_ref, wu2_ref, bu2_ref,
             ws1_ref, bs1_ref, ems_ref, wds_ref, bsd_ref, ws2_ref, bs2_ref,
             upmat_ref, o_ref):
        f32 = jnp.float32

        x0 = x0_ref[...].reshape(nb * chA, S0)
        y = jnp.dot(wu1_ref[...], x0, preferred_element_type=f32) + bu1_ref[...]
        y = jnp.maximum(y, 0.0)
        y = dw3x3(y, emu_ref, wdu_ref, W0, S0) + bud_ref[...]
        y = jnp.dot(wu2_ref[...], y, preferred_element_type=f32) + bu2_ref[...]
        y = jnp.maximum(y, 0.0)
        up = jnp.dot(y, upmat_ref[...], preferred_element_type=f32)

        x1 = x1_ref[...].reshape(nb * chB, S1)
        z = jnp.dot(ws1_ref[...], x1, preferred_element_type=f32) + bs1_ref[...]
        z = jnp.maximum(z, 0.0)
        z = dw3x3(z, ems_ref, wds_ref, W1, S1) + bsd_ref[...]
        z = jnp.dot(ws2_ref[...], z, preferred_element_type=f32) + bs2_ref[...]
        z = jnp.maximum(z, 0.0)

        o_ref[:, :C, :] = up.reshape(nb, C, S1)
        o_ref[:, C:, :] = z.reshape(nb, C, S1)

    return body


def kernel(x0, x1, up_w1, up_b1, up_wdw, up_bdw, up_w2, up_b2,
           sh_w1, sh_b1, sh_wdw, sh_bdw, sh_w2, sh_b2):
    N, chA, H0, W0 = x0.shape
    _, chB, H1, W1 = x1.shape
    C = up_w1.shape[0]
    S0, S1 = H0 * W0, H1 * W1
    nb = NB if N % NB == 0 else 1

    x0f = x0.reshape(N, chA, S0)
    x1f = x1.reshape(N, chB, S1)

    eye = jnp.eye(nb, dtype=jnp.float32)

    def bd(w):
        return jnp.kron(eye, w)

    def tl(b):
        return jnp.tile(b, (nb, 1))

    wu1, wu2 = bd(up_w1), bd(up_w2)
    ws1, ws2 = bd(sh_w1), bd(sh_w2)
    bu1, bud, bu2 = tl(up_b1), tl(up_bdw), tl(up_b2)
    bs1, bsd, bs2 = tl(sh_b1), tl(sh_bdw), tl(sh_b2)

    emu = jnp.asarray(_edge_masks(H0, W0))
    ems = jnp.asarray(_edge_masks(H1, W1))
    upmat = jnp.asarray(_up2x_matrix(H0, W0))

    body = _make_body(nb, chA, chB, C, W0, W1, S0, S1)

    def cspec(arr):
        nd = arr.ndim
        return pl.BlockSpec(arr.shape, lambda b, _nd=nd: (0,) * _nd)

    out = pl.pallas_call(
        body,
        out_shape=jax.ShapeDtypeStruct((N, 2 * C, S1), jnp.float32),
        grid=(N // nb,),
        in_specs=[
            pl.BlockSpec((nb, chA, S0), lambda b: (b, 0, 0)),
            pl.BlockSpec((nb, chB, S1), lambda b: (b, 0, 0)),
            cspec(wu1), cspec(bu1), cspec(emu), cspec(up_wdw), cspec(bud),
            cspec(wu2), cspec(bu2),
            cspec(ws1), cspec(bs1), cspec(ems), cspec(sh_wdw), cspec(bsd),
            cspec(ws2), cspec(bs2),
            cspec(upmat),
        ],
        out_specs=pl.BlockSpec((nb, 2 * C, S1), lambda b: (b, 0, 0)),
        compiler_params=pltpu.CompilerParams(
            dimension_semantics=("parallel",),
            vmem_limit_bytes=100 * 1024 * 1024,
        ),
    )(x0f, x1f, wu1, bu1, emu, up_wdw, bud, wu2, bu2,
      ws1, bs1, ems, sh_wdw, bsd, ws2, bs2, upmat)

    return out.reshape(N, 2 * C, H1, W1)
